# speedup vs baseline: 9.3545x; 9.3545x over previous
import numpy as np
from math import comb, factorial, floor

import jax
import jax.numpy as jnp
from jax.experimental import pallas as pl

_MAX_L = 4


def _tables(max_l):
    dst, pows, cs, ns = [], [], [], []
    for l in range(max_l + 1):
        for m in range(-l, l + 1):
            am = abs(m)
            n_lm = (1.0 / (2.0 ** am * factorial(l))) * np.sqrt(
                2.0 * factorial(l + am) * factorial(l - am) / (2.0 if m == 0 else 1.0))
            ns.append(n_lm)
            vm = 0.5 if m < 0 else 0.0
            vmax = floor(am / 2.0 - vm) + vm
            for t in range(0, (l - am) // 2 + 1):
                for u in range(0, t + 1):
                    v = vm
                    while v <= vmax + 1e-9:
                        c = ((-1.0) ** int(round(t + v - vm))) * (0.25 ** t) \
                            * comb(l, t) * comb(l - t, am + t) * comb(t, u) * comb(am, int(round(2 * v)))
                        dst.append(l * (l + 1) + m)
                        pows.append([int(round(2 * t + am - 2 * (u + v))),
                                     int(round(2 * (u + v))),
                                     int(l - 2 * t - am)])
                        cs.append(c)
                        v += 1.0
    return dst, pows, cs, ns


def _build_matrix():
    dst, pows, cs, ns = _tables(_MAX_L)
    monos = sorted({tuple(p) for p in pows})
    midx = {m: i for i, m in enumerate(monos)}
    n_out = len(ns)
    mat = np.zeros((len(monos), n_out), dtype=np.float64)
    for d, p, c in zip(dst, pows, cs):
        mat[midx[tuple(p)], d] += c
    mat = mat * np.asarray(ns, dtype=np.float64)[None, :]
    return monos, mat.astype(np.float32)


_MONOS, _MAT = _build_matrix()
_N_MONO = len(_MONOS)
_N_OUT = _MAT.shape[1]

_BLOCK = 3200


def _body(xt_ref, m_ref, o_ref):
    x = xt_ref[0:1, :]
    y = xt_ref[1:2, :]
    z = xt_ref[2:3, :]
    xp = [None] * 5
    yp = [None] * 5
    zp = [None] * 5
    xp[1], xp[2] = x, x * x
    yp[1], yp[2] = y, y * y
    zp[1], zp[2] = z, z * z
    xp[3], xp[4] = xp[2] * x, xp[2] * xp[2]
    yp[3], yp[4] = yp[2] * y, yp[2] * yp[2]
    zp[3], zp[4] = zp[2] * z, zp[2] * zp[2]
    rows = []
    for (a, b, c) in _MONOS:
        facs = []
        if a:
            facs.append(xp[a])
        if b:
            facs.append(yp[b])
        if c:
            facs.append(zp[c])
        if not facs:
            v = jnp.ones_like(x)
        else:
            v = facs[0]
            for f in facs[1:]:
                v = v * f
        rows.append(v)
    p = jnp.concatenate(rows, axis=0)
    o_ref[...] = jax.lax.dot_general(
        p, m_ref[...], (((0,), (0,)), ((), ())),
        preferred_element_type=jnp.float32,
        precision=jax.lax.Precision.HIGHEST)


def kernel(xyz):
    in_shape = xyz.shape
    x = xyz.reshape(-1, 3)
    n = x.shape[0]
    xt = x.T
    mat = jnp.asarray(_MAT)
    grid = n // _BLOCK
    out = pl.pallas_call(
        _body,
        grid=(grid,),
        in_specs=[
            pl.BlockSpec((3, _BLOCK), lambda i: (0, i)),
            pl.BlockSpec((_N_MONO, _N_OUT), lambda i: (0, 0)),
        ],
        out_specs=pl.BlockSpec((_BLOCK, _N_OUT), lambda i: (i, 0)),
        out_shape=jax.ShapeDtypeStruct((n, _N_OUT), jnp.float32),
    )(xt, mat)
    return out.reshape(*in_shape[:-1], _N_OUT)
